# Initial kernel scaffold; baseline (speedup 1.0000x reference)
import jax
import jax.numpy as jnp
from jax.experimental import pallas as pl


def kernel(input_tensor, rows, cols, vals, kernel):
    raise NotImplementedError("write your pallas kernel here")


# speedup vs baseline: 4.0062x; 4.0062x over previous
import functools

import jax
import jax.numpy as jnp
import numpy as np
from jax import lax
from jax.experimental import pallas as pl
from jax.experimental.pallas import tpu as pltpu
from jax.experimental.pallas import tpu_sc as plsc

M = 10000
MP = 10240
B = 8
F = 128
NSUB = 16
NCORE = 2
EDGE_BATCH = 128
BATCHES = 162
EPS = NSUB * BATCHES * EDGE_BATCH
STRIPE = MP // NSUB
ZROWS = 40
OROWS = 160

_C = np.array(
    [
        [1.0, -1.5, 0.75, -0.125],
        [0.0, 1.5, -1.5, 0.375],
        [0.0, 0.0, 0.75, -0.375],
        [0.0, 0.0, 0.09375, -0.046875],
    ],
    dtype=np.float32,
)


def _spmv_body(x_hbm, rows_hbm, cols_hbm, vals_hbm, y_hbm,
               cidx, ridx, vv, gbuf, zbuf, obuf, acc, sem):
    c = lax.axis_index("c")
    s = lax.axis_index("s")

    def zero_row(r, carry):
        zeros = jnp.zeros((16,), jnp.float32)
        for t in range(F // 16):
            zbuf[r, pl.ds(t * 16, 16)] = zeros
        return carry

    lax.fori_loop(0, ZROWS, zero_row, None)

    for b in range(B):
        @pl.when(c == b // (B // NCORE))
        def _():
            for j in range(STRIPE // ZROWS):
                pltpu.sync_copy(zbuf, acc.at[pl.ds(s * STRIPE + j * ZROWS, ZROWS)])
            plsc.subcore_barrier()

            def batch_body(i, carry):
                base = s * (BATCHES * EDGE_BATCH) + i * EDGE_BATCH
                pltpu.sync_copy(cols_hbm.at[pl.ds(base, EDGE_BATCH)], cidx)
                pltpu.sync_copy(rows_hbm.at[pl.ds(base, EDGE_BATCH)], ridx)
                pltpu.sync_copy(vals_hbm.at[pl.ds(base, EDGE_BATCH)], vv)
                for t in range(EDGE_BATCH // 16):
                    cidx[pl.ds(t * 16, 16)] = cidx[pl.ds(t * 16, 16)] + (b * MP)
                pltpu.async_copy(x_hbm.at[cidx], gbuf, sem).wait()

                def scale_body(g, carry2):
                    vvec = vv[pl.ds(g * 16, 16)]
                    for j in range(16):
                        v = vvec[j]
                        e = g * 16 + j
                        for t in range(F // 16):
                            gbuf[e, pl.ds(t * 16, 16)] = (
                                gbuf[e, pl.ds(t * 16, 16)] * v
                            )
                    return carry2

                lax.fori_loop(0, EDGE_BATCH // 16, scale_body, None)
                pltpu.sync_copy(gbuf, acc.at[ridx], add=True)
                return carry

            lax.fori_loop(0, BATCHES, batch_body, None)
            plsc.subcore_barrier()
            for q in range(STRIPE // OROWS):
                pltpu.sync_copy(acc.at[pl.ds(s * STRIPE + q * OROWS, OROWS)], obuf)
                pltpu.sync_copy(
                    obuf,
                    y_hbm.at[pl.ds(b * MP + s * STRIPE + q * OROWS, OROWS)],
                )
            plsc.subcore_barrier()


_spmv = functools.partial(
    pl.kernel,
    mesh=plsc.VectorSubcoreMesh(core_axis_name="c", subcore_axis_name="s"),
    out_type=jax.ShapeDtypeStruct((B * MP, F), jnp.float32),
    scratch_types=[
        pltpu.VMEM((EDGE_BATCH,), jnp.int32),
        pltpu.VMEM((EDGE_BATCH,), jnp.int32),
        pltpu.VMEM((EDGE_BATCH,), jnp.float32),
        pltpu.VMEM((EDGE_BATCH, F), jnp.float32),
        pltpu.VMEM((ZROWS, F), jnp.float32),
        pltpu.VMEM((OROWS, F), jnp.float32),
        pltpu.VMEM_SHARED((MP, F), jnp.float32),
        pltpu.SemaphoreType.DMA,
    ],
)(_spmv_body)


def _combine_body(x_ref, y1_ref, y2_ref, y3_ref, w_ref, o_ref):
    acc = jnp.dot(x_ref[0], w_ref[pl.ds(0, F), :],
                  preferred_element_type=jnp.float32)
    acc += jnp.dot(y1_ref[0], w_ref[pl.ds(F, F), :],
                   preferred_element_type=jnp.float32)
    acc += jnp.dot(y2_ref[0], w_ref[pl.ds(2 * F, F), :],
                   preferred_element_type=jnp.float32)
    acc += jnp.dot(y3_ref[0], w_ref[pl.ds(3 * F, F), :],
                   preferred_element_type=jnp.float32)
    o_ref[0] = acc


def kernel(input_tensor, rows, cols, vals, kernel):
    E = rows.shape[0]
    pad = EPS - E
    rows_p = jnp.concatenate([rows, jnp.zeros((pad,), jnp.int32)])
    cols_p = jnp.concatenate([cols, jnp.zeros((pad,), jnp.int32)])
    vals_p = jnp.concatenate([vals, jnp.zeros((pad,), jnp.float32)])

    x0 = jnp.pad(input_tensor, ((0, 0), (0, MP - M), (0, 0))).reshape(B * MP, F)
    y1 = _spmv(x0, rows_p, cols_p, vals_p)
    y2 = _spmv(y1, rows_p, cols_p, vals_p)
    y3 = _spmv(y2, rows_p, cols_p, vals_p)

    wr = kernel.reshape(F, 4, -1)
    wp = jnp.einsum("kj,fko->jfo", jnp.asarray(_C), wr)
    wcat = wp.reshape(4 * F, -1)
    fout = wcat.shape[1]

    blk = 2000
    bspec = pl.BlockSpec((1, blk, F), lambda b, i: (b, i, 0))
    out = pl.pallas_call(
        _combine_body,
        grid=(B, M // blk),
        in_specs=[
            bspec,
            bspec,
            bspec,
            bspec,
            pl.BlockSpec((4 * F, fout), lambda b, i: (0, 0)),
        ],
        out_specs=pl.BlockSpec((1, blk, fout), lambda b, i: (b, i, 0)),
        out_shape=jax.ShapeDtypeStruct((B, M, fout), jnp.float32),
    )(
        x0.reshape(B, MP, F),
        y1.reshape(B, MP, F),
        y2.reshape(B, MP, F),
        y3.reshape(B, MP, F),
        wcat,
    )
    return out


# speedup vs baseline: 5.6658x; 1.4142x over previous
import functools

import jax
import jax.numpy as jnp
import numpy as np
from jax import lax
from jax.experimental import pallas as pl
from jax.experimental.pallas import tpu as pltpu
from jax.experimental.pallas import tpu_sc as plsc

M = 10000
MP = 10240
B = 8
F = 128
NSUB = 16
NCORE = 2
EDGE_BATCH = 128
BATCHES = 162
EPS = NSUB * BATCHES * EDGE_BATCH
STRIPE = MP // NSUB
ZROWS = 40

_C = np.array(
    [
        [1.0, -1.5, 0.75, -0.125],
        [0.0, 1.5, -1.5, 0.375],
        [0.0, 0.0, 0.75, -0.375],
        [0.0, 0.0, 0.09375, -0.046875],
    ],
    dtype=np.float32,
)


def _spmv_body(x_hbm, rows_hbm, cols_hbm, vals_hbm, y_hbm,
               cidx0, ridx0, vv0, gbuf0, semg0, semsc0,
               cidx1, ridx1, vv1, gbuf1, semg1, semsc1,
               zbuf, acc):
    c = lax.axis_index("c")
    s = lax.axis_index("s")
    slots = (
        (cidx0, ridx0, vv0, gbuf0, semg0, semsc0),
        (cidx1, ridx1, vv1, gbuf1, semg1, semsc1),
    )

    def b_body(bi, bcarry):
        cbase = (c * (B // NCORE) + bi) * MP
        if True:

            def loadidx(k, p):
                cidx, ridx, vv, _, _, _ = slots[p]
                base = s * (BATCHES * EDGE_BATCH) + k * EDGE_BATCH
                pltpu.sync_copy(cols_hbm.at[pl.ds(base, EDGE_BATCH)], cidx)
                pltpu.sync_copy(rows_hbm.at[pl.ds(base, EDGE_BATCH)], ridx)
                pltpu.sync_copy(vals_hbm.at[pl.ds(base, EDGE_BATCH)], vv)
                for t in range(EDGE_BATCH // 16):
                    cidx[pl.ds(t * 16, 16)] = cidx[pl.ds(t * 16, 16)] + cbase

            def fire_gather(p):
                cidx, _, _, gbuf, semg, _ = slots[p]
                pltpu.async_copy(x_hbm.at[cidx], gbuf, semg)

            def wait_gather(p):
                cidx, _, _, gbuf, semg, _ = slots[p]
                pltpu.make_async_copy(x_hbm.at[cidx], gbuf, semg).wait()

            def scale(p):
                _, _, vv, gbuf, _, _ = slots[p]

                def scale_body(g, carry):
                    vvec = vv[pl.ds(g * 16, 16)]
                    for j in range(16):
                        v = vvec[j]
                        e = g * 16 + j
                        for t in range(F // 16):
                            gbuf[e, pl.ds(t * 16, 16)] = (
                                gbuf[e, pl.ds(t * 16, 16)] * v
                            )
                    return carry

                lax.fori_loop(0, EDGE_BATCH // 16, scale_body, None)

            def fire_scatter(p):
                _, ridx, _, gbuf, _, semsc = slots[p]
                pltpu.async_copy(gbuf, acc.at[ridx], semsc, add=True)

            def wait_scatter(p):
                _, ridx, _, gbuf, _, semsc = slots[p]
                pltpu.make_async_copy(gbuf, acc.at[ridx], semsc).wait()

            def zero_row(r, carry):
                zeros = jnp.zeros((16,), jnp.float32)
                for t in range(F // 16):
                    zbuf[r, pl.ds(t * 16, 16)] = zeros
                return carry

            lax.fori_loop(0, ZROWS, zero_row, None)
            for j in range(STRIPE // ZROWS):
                pltpu.sync_copy(zbuf, acc.at[pl.ds(s * STRIPE + j * ZROWS, ZROWS)])
            plsc.subcore_barrier()

            loadidx(0, 0)
            fire_gather(0)
            loadidx(1, 1)
            fire_gather(1)
            wait_gather(0)
            scale(0)
            fire_scatter(0)
            wait_scatter(0)
            loadidx(2, 0)
            fire_gather(0)
            wait_gather(1)
            scale(1)
            fire_scatter(1)

            def steady(i, carry):
                wait_scatter(1)
                loadidx(2 * i + 1, 1)
                fire_gather(1)
                wait_gather(0)
                scale(0)
                fire_scatter(0)
                wait_scatter(0)
                loadidx(2 * i + 2, 0)
                fire_gather(0)
                wait_gather(1)
                scale(1)
                fire_scatter(1)
                return carry

            lax.fori_loop(1, BATCHES // 2 - 1, steady, None)
            wait_scatter(1)
            loadidx(BATCHES - 1, 1)
            fire_gather(1)
            wait_gather(0)
            scale(0)
            fire_scatter(0)
            wait_gather(1)
            scale(1)
            fire_scatter(1)
            wait_scatter(0)
            wait_scatter(1)
            plsc.subcore_barrier()

            for q in range(STRIPE // ZROWS):
                pltpu.sync_copy(acc.at[pl.ds(s * STRIPE + q * ZROWS, ZROWS)], zbuf)
                pltpu.sync_copy(
                    zbuf,
                    y_hbm.at[pl.ds(cbase + s * STRIPE + q * ZROWS, ZROWS)],
                )
            plsc.subcore_barrier()
        return bcarry

    lax.fori_loop(0, B // NCORE, b_body, None)


def _slot_scratch():
    return [
        pltpu.VMEM((EDGE_BATCH,), jnp.int32),
        pltpu.VMEM((EDGE_BATCH,), jnp.int32),
        pltpu.VMEM((EDGE_BATCH,), jnp.float32),
        pltpu.VMEM((EDGE_BATCH, F), jnp.float32),
        pltpu.SemaphoreType.DMA,
        pltpu.SemaphoreType.DMA,
    ]


_spmv = functools.partial(
    pl.kernel,
    mesh=plsc.VectorSubcoreMesh(core_axis_name="c", subcore_axis_name="s"),
    out_type=jax.ShapeDtypeStruct((B * MP, F), jnp.float32),
    scratch_types=_slot_scratch() + _slot_scratch() + [
        pltpu.VMEM((ZROWS, F), jnp.float32),
        pltpu.VMEM_SHARED((MP, F), jnp.float32),
    ],
)(_spmv_body)


def _combine_body(x_ref, y1_ref, y2_ref, y3_ref, w_ref, o_ref):
    acc = jnp.dot(x_ref[0], w_ref[pl.ds(0, F), :],
                  preferred_element_type=jnp.float32)
    acc += jnp.dot(y1_ref[0], w_ref[pl.ds(F, F), :],
                   preferred_element_type=jnp.float32)
    acc += jnp.dot(y2_ref[0], w_ref[pl.ds(2 * F, F), :],
                   preferred_element_type=jnp.float32)
    acc += jnp.dot(y3_ref[0], w_ref[pl.ds(3 * F, F), :],
                   preferred_element_type=jnp.float32)
    o_ref[0] = acc


def kernel(input_tensor, rows, cols, vals, kernel):
    E = rows.shape[0]
    pad = EPS - E
    rows_p = jnp.concatenate([rows, jnp.zeros((pad,), jnp.int32)])
    cols_p = jnp.concatenate([cols, jnp.zeros((pad,), jnp.int32)])
    vals_p = jnp.concatenate([vals, jnp.zeros((pad,), jnp.float32)])

    x0 = jnp.pad(input_tensor, ((0, 0), (0, MP - M), (0, 0))).reshape(B * MP, F)
    y1 = _spmv(x0, rows_p, cols_p, vals_p)
    y2 = _spmv(y1, rows_p, cols_p, vals_p)
    y3 = _spmv(y2, rows_p, cols_p, vals_p)

    wr = kernel.reshape(F, 4, -1)
    wp = jnp.einsum("kj,fko->jfo", jnp.asarray(_C), wr)
    wcat = wp.reshape(4 * F, -1)
    fout = wcat.shape[1]

    blk = 2000
    bspec = pl.BlockSpec((1, blk, F), lambda b, i: (b, i, 0))
    out = pl.pallas_call(
        _combine_body,
        grid=(B, M // blk),
        in_specs=[
            bspec,
            bspec,
            bspec,
            bspec,
            pl.BlockSpec((4 * F, fout), lambda b, i: (0, 0)),
        ],
        out_specs=pl.BlockSpec((1, blk, fout), lambda b, i: (b, i, 0)),
        out_shape=jax.ShapeDtypeStruct((B, M, fout), jnp.float32),
    )(
        x0.reshape(B, MP, F),
        y1.reshape(B, MP, F),
        y2.reshape(B, MP, F),
        y3.reshape(B, MP, F),
        wcat,
    )
    return out


# speedup vs baseline: 7.0269x; 1.2402x over previous
import functools

import jax
import jax.numpy as jnp
import numpy as np
from jax import lax
from jax.experimental import pallas as pl
from jax.experimental.pallas import tpu as pltpu
from jax.experimental.pallas import tpu_sc as plsc

M = 10000
MP = 10240
B = 8
F = 128
NSUB = 16
NCORE = 2
EDGE_BATCH = 160
BATCHES = 130
EPS = NSUB * BATCHES * EDGE_BATCH
STRIPE = MP // NSUB
ZROWS = 40

_C = np.array(
    [
        [1.0, -1.5, 0.75, -0.125],
        [0.0, 1.5, -1.5, 0.375],
        [0.0, 0.0, 0.75, -0.375],
        [0.0, 0.0, 0.09375, -0.046875],
    ],
    dtype=np.float32,
)


def _spmv_body(x_hbm, rows_hbm, cols_hbm, vals_hbm, y_hbm,
               cidx0, ridx0, vv0, gbuf0, semg0, semsc0, semi0,
               cidx1, ridx1, vv1, gbuf1, semg1, semsc1, semi1,
               zbuf, acc):
    c = lax.axis_index("c")
    s = lax.axis_index("s")
    slots = (
        (cidx0, ridx0, vv0, gbuf0, semg0, semsc0, semi0),
        (cidx1, ridx1, vv1, gbuf1, semg1, semsc1, semi1),
    )

    def b_body(bi, bcarry):
        cbase = (c * (B // NCORE) + bi) * MP
        if True:

            def loadidx(k, p):
                cidx, ridx, vv, _, _, _, semi = slots[p]
                base = s * (BATCHES * EDGE_BATCH) + k * EDGE_BATCH
                cs = cols_hbm.at[pl.ds(base, EDGE_BATCH)]
                rs = rows_hbm.at[pl.ds(base, EDGE_BATCH)]
                vs = vals_hbm.at[pl.ds(base, EDGE_BATCH)]
                pltpu.async_copy(cs, cidx, semi)
                pltpu.async_copy(rs, ridx, semi)
                pltpu.async_copy(vs, vv, semi)
                pltpu.make_async_copy(cs, cidx, semi).wait()
                pltpu.make_async_copy(rs, ridx, semi).wait()
                pltpu.make_async_copy(vs, vv, semi).wait()
                for t in range(EDGE_BATCH // 16):
                    cidx[pl.ds(t * 16, 16)] = cidx[pl.ds(t * 16, 16)] + cbase

            def fire_gather(p):
                cidx, _, _, gbuf, semg, _, _ = slots[p]
                pltpu.async_copy(x_hbm.at[cidx], gbuf, semg)

            def wait_gather(p):
                cidx, _, _, gbuf, semg, _, _ = slots[p]
                pltpu.make_async_copy(x_hbm.at[cidx], gbuf, semg).wait()

            def scale(p):
                _, _, vv, gbuf, _, _, _ = slots[p]

                def scale_body(g, carry):
                    vvec = vv[pl.ds(g * 16, 16)]
                    for j in range(16):
                        v = vvec[j]
                        e = g * 16 + j
                        for t in range(F // 16):
                            gbuf[e, pl.ds(t * 16, 16)] = (
                                gbuf[e, pl.ds(t * 16, 16)] * v
                            )
                    return carry

                lax.fori_loop(0, EDGE_BATCH // 16, scale_body, None)

            def fire_scatter(p):
                _, ridx, _, gbuf, _, semsc, _ = slots[p]
                pltpu.async_copy(gbuf, acc.at[ridx], semsc, add=True)

            def wait_scatter(p):
                _, ridx, _, gbuf, _, semsc, _ = slots[p]
                pltpu.make_async_copy(gbuf, acc.at[ridx], semsc).wait()

            def zero_row(r, carry):
                zeros = jnp.zeros((16,), jnp.float32)
                for t in range(F // 16):
                    zbuf[r, pl.ds(t * 16, 16)] = zeros
                return carry

            lax.fori_loop(0, ZROWS, zero_row, None)
            for j in range(STRIPE // ZROWS):
                pltpu.sync_copy(zbuf, acc.at[pl.ds(s * STRIPE + j * ZROWS, ZROWS)])
            plsc.subcore_barrier()

            loadidx(0, 0)
            fire_gather(0)
            loadidx(1, 1)
            fire_gather(1)
            wait_gather(0)
            scale(0)
            fire_scatter(0)
            wait_scatter(0)
            loadidx(2, 0)
            fire_gather(0)
            wait_gather(1)
            scale(1)
            fire_scatter(1)

            def steady(i, carry):
                wait_scatter(1)
                loadidx(2 * i + 1, 1)
                fire_gather(1)
                wait_gather(0)
                scale(0)
                fire_scatter(0)
                wait_scatter(0)
                loadidx(2 * i + 2, 0)
                fire_gather(0)
                wait_gather(1)
                scale(1)
                fire_scatter(1)
                return carry

            lax.fori_loop(1, BATCHES // 2 - 1, steady, None)
            wait_scatter(1)
            loadidx(BATCHES - 1, 1)
            fire_gather(1)
            wait_gather(0)
            scale(0)
            fire_scatter(0)
            wait_gather(1)
            scale(1)
            fire_scatter(1)
            wait_scatter(0)
            wait_scatter(1)
            plsc.subcore_barrier()

            for q in range(STRIPE // ZROWS):
                pltpu.sync_copy(acc.at[pl.ds(s * STRIPE + q * ZROWS, ZROWS)], zbuf)
                pltpu.sync_copy(
                    zbuf,
                    y_hbm.at[pl.ds(cbase + s * STRIPE + q * ZROWS, ZROWS)],
                )
            plsc.subcore_barrier()
        return bcarry

    lax.fori_loop(0, B // NCORE, b_body, None)


def _slot_scratch():
    return [
        pltpu.VMEM((EDGE_BATCH,), jnp.int32),
        pltpu.VMEM((EDGE_BATCH,), jnp.int32),
        pltpu.VMEM((EDGE_BATCH,), jnp.float32),
        pltpu.VMEM((EDGE_BATCH, F), jnp.float32),
        pltpu.SemaphoreType.DMA,
        pltpu.SemaphoreType.DMA,
        pltpu.SemaphoreType.DMA,
    ]


_spmv = functools.partial(
    pl.kernel,
    mesh=plsc.VectorSubcoreMesh(core_axis_name="c", subcore_axis_name="s"),
    out_type=jax.ShapeDtypeStruct((B * MP, F), jnp.float32),
    scratch_types=_slot_scratch() + _slot_scratch() + [
        pltpu.VMEM((ZROWS, F), jnp.float32),
        pltpu.VMEM_SHARED((MP, F), jnp.float32),
    ],
)(_spmv_body)


def _combine_body(x_ref, y1_ref, y2_ref, y3_ref, w_ref, o_ref):
    acc = jnp.dot(x_ref[0], w_ref[pl.ds(0, F), :],
                  preferred_element_type=jnp.float32)
    acc += jnp.dot(y1_ref[0], w_ref[pl.ds(F, F), :],
                   preferred_element_type=jnp.float32)
    acc += jnp.dot(y2_ref[0], w_ref[pl.ds(2 * F, F), :],
                   preferred_element_type=jnp.float32)
    acc += jnp.dot(y3_ref[0], w_ref[pl.ds(3 * F, F), :],
                   preferred_element_type=jnp.float32)
    o_ref[0] = acc


def kernel(input_tensor, rows, cols, vals, kernel):
    E = rows.shape[0]
    pad = EPS - E
    rows_p = jnp.concatenate([rows, jnp.zeros((pad,), jnp.int32)])
    cols_p = jnp.concatenate([cols, jnp.zeros((pad,), jnp.int32)])
    vals_p = jnp.concatenate([vals, jnp.zeros((pad,), jnp.float32)])

    x0 = jnp.pad(input_tensor, ((0, 0), (0, MP - M), (0, 0))).reshape(B * MP, F)
    y1 = _spmv(x0, rows_p, cols_p, vals_p)
    y2 = _spmv(y1, rows_p, cols_p, vals_p)
    y3 = _spmv(y2, rows_p, cols_p, vals_p)

    wr = kernel.reshape(F, 4, -1)
    wp = jnp.einsum("kj,fko->jfo", jnp.asarray(_C), wr)
    wcat = wp.reshape(4 * F, -1)
    fout = wcat.shape[1]

    blk = 2000
    bspec = pl.BlockSpec((1, blk, F), lambda b, i: (b, i, 0))
    out = pl.pallas_call(
        _combine_body,
        grid=(B, M // blk),
        in_specs=[
            bspec,
            bspec,
            bspec,
            bspec,
            pl.BlockSpec((4 * F, fout), lambda b, i: (0, 0)),
        ],
        out_specs=pl.BlockSpec((1, blk, fout), lambda b, i: (b, i, 0)),
        out_shape=jax.ShapeDtypeStruct((B, M, fout), jnp.float32),
    )(
        x0.reshape(B, MP, F),
        y1.reshape(B, MP, F),
        y2.reshape(B, MP, F),
        y3.reshape(B, MP, F),
        wcat,
    )
    return out


# speedup vs baseline: 7.1490x; 1.0174x over previous
import functools

import jax
import jax.numpy as jnp
import numpy as np
from jax import lax
from jax.experimental import pallas as pl
from jax.experimental.pallas import tpu as pltpu
from jax.experimental.pallas import tpu_sc as plsc

M = 10000
MP = 10240
B = 8
F = 128
NSUB = 16
NCORE = 2
EDGE_BATCH = 160
BATCHES = 130
EPS = NSUB * BATCHES * EDGE_BATCH
STRIPE = MP // NSUB
ZROWS = 40

_C = np.array(
    [
        [1.0, -1.5, 0.75, -0.125],
        [0.0, 1.5, -1.5, 0.375],
        [0.0, 0.0, 0.75, -0.375],
        [0.0, 0.0, 0.09375, -0.046875],
    ],
    dtype=np.float32,
)


def _spmv_body(x_hbm, rows_hbm, cols_hbm, vals_hbm, y_hbm,
               cidx0, ridx0, vv0, gbuf0, semg0, semsc0, semi0,
               cidx1, ridx1, vv1, gbuf1, semg1, semsc1, semi1,
               zbuf, acc):
    c = lax.axis_index("c")
    s = lax.axis_index("s")
    slots = (
        (cidx0, ridx0, vv0, gbuf0, semg0, semsc0, semi0),
        (cidx1, ridx1, vv1, gbuf1, semg1, semsc1, semi1),
    )

    def b_body(bi, bcarry):
        cbase = (c * (B // NCORE) + bi) * MP
        if True:

            def prefetch(k, p, drain_scatter):
                cidx, ridx, vv, _, _, _, semi = slots[p]
                base = s * (BATCHES * EDGE_BATCH) + k * EDGE_BATCH
                cs = cols_hbm.at[pl.ds(base, EDGE_BATCH)]
                rs = rows_hbm.at[pl.ds(base, EDGE_BATCH)]
                vs = vals_hbm.at[pl.ds(base, EDGE_BATCH)]
                pltpu.async_copy(cs, cidx, semi)
                pltpu.async_copy(vs, vv, semi)
                if drain_scatter:
                    wait_scatter(p)
                pltpu.async_copy(rs, ridx, semi)
                pltpu.make_async_copy(cs, cidx, semi).wait()
                for t in range(EDGE_BATCH // 16):
                    cidx[pl.ds(t * 16, 16)] = cidx[pl.ds(t * 16, 16)] + cbase
                pltpu.make_async_copy(vs, vv, semi).wait()
                pltpu.make_async_copy(rs, ridx, semi).wait()
                fire_gather(p)

            def fire_gather(p):
                cidx, _, _, gbuf, semg, _, _ = slots[p]
                pltpu.async_copy(x_hbm.at[cidx], gbuf, semg)

            def wait_gather(p):
                cidx, _, _, gbuf, semg, _, _ = slots[p]
                pltpu.make_async_copy(x_hbm.at[cidx], gbuf, semg).wait()

            def scale(p):
                _, _, vv, gbuf, _, _, _ = slots[p]

                def scale_body(g, carry):
                    vvec = vv[pl.ds(g * 16, 16)]
                    for j in range(16):
                        v = vvec[j]
                        e = g * 16 + j
                        for t in range(F // 16):
                            gbuf[e, pl.ds(t * 16, 16)] = (
                                gbuf[e, pl.ds(t * 16, 16)] * v
                            )
                    return carry

                lax.fori_loop(0, EDGE_BATCH // 16, scale_body, None)

            def fire_scatter(p):
                _, ridx, _, gbuf, _, semsc, _ = slots[p]
                pltpu.async_copy(gbuf, acc.at[ridx], semsc, add=True)

            def wait_scatter(p):
                _, ridx, _, gbuf, _, semsc, _ = slots[p]
                pltpu.make_async_copy(gbuf, acc.at[ridx], semsc).wait()

            def zero_row(r, carry):
                zeros = jnp.zeros((16,), jnp.float32)
                for t in range(F // 16):
                    zbuf[r, pl.ds(t * 16, 16)] = zeros
                return carry

            lax.fori_loop(0, ZROWS, zero_row, None)
            for j in range(STRIPE // ZROWS):
                pltpu.sync_copy(zbuf, acc.at[pl.ds(s * STRIPE + j * ZROWS, ZROWS)])
            plsc.subcore_barrier()

            prefetch(0, 0, False)
            prefetch(1, 1, False)
            wait_gather(0)
            scale(0)
            fire_scatter(0)
            prefetch(2, 0, True)
            wait_gather(1)
            scale(1)
            fire_scatter(1)

            def steady(i, carry):
                prefetch(2 * i + 1, 1, True)
                wait_gather(0)
                scale(0)
                fire_scatter(0)
                prefetch(2 * i + 2, 0, True)
                wait_gather(1)
                scale(1)
                fire_scatter(1)
                return carry

            lax.fori_loop(1, BATCHES // 2 - 1, steady, None)
            prefetch(BATCHES - 1, 1, True)
            wait_gather(0)
            scale(0)
            fire_scatter(0)
            wait_gather(1)
            scale(1)
            fire_scatter(1)
            wait_scatter(0)
            wait_scatter(1)
            plsc.subcore_barrier()

            for q in range(STRIPE // ZROWS):
                pltpu.sync_copy(acc.at[pl.ds(s * STRIPE + q * ZROWS, ZROWS)], zbuf)
                pltpu.sync_copy(
                    zbuf,
                    y_hbm.at[pl.ds(cbase + s * STRIPE + q * ZROWS, ZROWS)],
                )
            plsc.subcore_barrier()
        return bcarry

    lax.fori_loop(0, B // NCORE, b_body, None)


def _slot_scratch():
    return [
        pltpu.VMEM((EDGE_BATCH,), jnp.int32),
        pltpu.VMEM((EDGE_BATCH,), jnp.int32),
        pltpu.VMEM((EDGE_BATCH,), jnp.float32),
        pltpu.VMEM((EDGE_BATCH, F), jnp.float32),
        pltpu.SemaphoreType.DMA,
        pltpu.SemaphoreType.DMA,
        pltpu.SemaphoreType.DMA,
    ]


_spmv = functools.partial(
    pl.kernel,
    mesh=plsc.VectorSubcoreMesh(core_axis_name="c", subcore_axis_name="s"),
    out_type=jax.ShapeDtypeStruct((B * MP, F), jnp.float32),
    scratch_types=_slot_scratch() + _slot_scratch() + [
        pltpu.VMEM((ZROWS, F), jnp.float32),
        pltpu.VMEM_SHARED((MP, F), jnp.float32),
    ],
)(_spmv_body)


def _combine_body(x_ref, y1_ref, y2_ref, y3_ref, w_ref, o_ref):
    acc = jnp.dot(x_ref[0], w_ref[pl.ds(0, F), :],
                  preferred_element_type=jnp.float32)
    acc += jnp.dot(y1_ref[0], w_ref[pl.ds(F, F), :],
                   preferred_element_type=jnp.float32)
    acc += jnp.dot(y2_ref[0], w_ref[pl.ds(2 * F, F), :],
                   preferred_element_type=jnp.float32)
    acc += jnp.dot(y3_ref[0], w_ref[pl.ds(3 * F, F), :],
                   preferred_element_type=jnp.float32)
    o_ref[0] = acc


def kernel(input_tensor, rows, cols, vals, kernel):
    E = rows.shape[0]
    pad = EPS - E
    rows_p = jnp.concatenate([rows, jnp.zeros((pad,), jnp.int32)])
    cols_p = jnp.concatenate([cols, jnp.zeros((pad,), jnp.int32)])
    vals_p = jnp.concatenate([vals, jnp.zeros((pad,), jnp.float32)])

    x0 = jnp.pad(input_tensor, ((0, 0), (0, MP - M), (0, 0))).reshape(B * MP, F)
    y1 = _spmv(x0, rows_p, cols_p, vals_p)
    y2 = _spmv(y1, rows_p, cols_p, vals_p)
    y3 = _spmv(y2, rows_p, cols_p, vals_p)

    wr = kernel.reshape(F, 4, -1)
    wp = jnp.einsum("kj,fko->jfo", jnp.asarray(_C), wr)
    wcat = wp.reshape(4 * F, -1)
    fout = wcat.shape[1]

    blk = 2000
    bspec = pl.BlockSpec((1, blk, F), lambda b, i: (b, i, 0))
    out = pl.pallas_call(
        _combine_body,
        grid=(B, M // blk),
        in_specs=[
            bspec,
            bspec,
            bspec,
            bspec,
            pl.BlockSpec((4 * F, fout), lambda b, i: (0, 0)),
        ],
        out_specs=pl.BlockSpec((1, blk, fout), lambda b, i: (b, i, 0)),
        out_shape=jax.ShapeDtypeStruct((B, M, fout), jnp.float32),
    )(
        x0.reshape(B, MP, F),
        y1.reshape(B, MP, F),
        y2.reshape(B, MP, F),
        y3.reshape(B, MP, F),
        wcat,
    )
    return out
